# baseline (speedup 1.0000x reference)
import functools

import jax
import jax.numpy as jnp
from jax import lax
from jax.experimental import pallas as pl
from jax.experimental.pallas import tpu as pltpu
from jax.experimental.pallas import tpu_sc as plsc

_NC = 2
_NS = 16
_NW = _NC * _NS
_LANES = 256
_ZCH = 64
_F32 = jnp.float32
_HI = lax.Precision.HIGHEST


def _ceil(a, b):
    return -(-a // b)


def _zero_acc(acc, zbuf, s, rps):
    base = s * rps
    nf, rem = rps // _ZCH, rps % _ZCH

    def zb(i, _):
        pltpu.sync_copy(zbuf, acc.at[pl.ds(base + i * _ZCH, _ZCH)])
        return _

    lax.fori_loop(0, nf, zb, None)
    if rem:
        pltpu.sync_copy(zbuf.at[pl.ds(0, rem)],
                        acc.at[pl.ds(base + nf * _ZCH, rem)])


_NBUF = 4
_PREF = 2


def _gs_loop(h_hbm, acc, idx_s, idx_d, bufs, sem_g, sem_s, rw):
    def wait_g(slot):
        pltpu.make_async_copy(h_hbm.at[idx_s.at[0]], bufs[slot], sem_g).wait()

    def wait_s():
        pltpu.make_async_copy(bufs[0], acc.at[idx_d.at[0]], sem_s).wait()

    def start_g(j, slot):
        pltpu.async_copy(h_hbm.at[idx_s.at[j]], bufs[slot], sem_g)

    def start_s(j, slot):
        pltpu.async_copy(bufs[slot], acc.at[idx_d.at[j]], sem_s, add=True)

    for j in range(_PREF):
        start_g(j, j)
    for j in range(_NBUF):
        wait_g(j % _NBUF)
        start_s(j, j % _NBUF)
        if j >= _PREF:
            wait_s()
        start_g(j + _PREF, (j + _PREF) % _NBUF)

    def body(g, _):
        j0 = g * _NBUF
        for b in range(_NBUF):
            wait_g(b)
            start_s(j0 + b, b)
            wait_s()
            start_g(j0 + b + _PREF, (b + _PREF) % _NBUF)
        return _

    lax.fori_loop(1, rw // _NBUF, body, None)

    for _e in range(_PREF):
        wait_g(0)
        wait_s()


def _load_idx(srcm, dstm, idx_s, idx_d, row0, rw):
    pltpu.sync_copy(srcm.at[pl.ds(row0, rw)], idx_s.at[pl.ds(0, rw)])
    for r in range(_PREF):
        for k in range(_LANES // 16):
            idx_s[rw + r, pl.ds(16 * k, 16)] = jnp.zeros((16,), jnp.int32)
    pltpu.sync_copy(dstm.at[pl.ds(row0, rw)], idx_d)


def _writeout(acc, out, c, s, n):
    ch = 8 * _ceil(n, _NS * 8)
    last = n - (_NS - 1) * ch

    @pl.when(s < _NS - 1)
    def _():
        pltpu.sync_copy(acc.at[pl.ds(s * ch, ch)],
                        out.at[c, pl.ds(s * ch, ch)])

    @pl.when(s == _NS - 1)
    def _():
        pltpu.sync_copy(acc.at[pl.ds((_NS - 1) * ch, last)],
                        out.at[c, pl.ds((_NS - 1) * ch, last)])


@functools.lru_cache(maxsize=None)
def _make_sc_agg3(N, rw, F):
    nacc = 128 * _ceil(N + 1, 128)
    rpsz = nacc // 16
    mesh = plsc.VectorSubcoreMesh(core_axis_name="c", subcore_axis_name="s",
                                  num_cores=_NC, num_subcores=_NS)

    def body(h0, sm0, dm0, h1, sm1, dm1, h2, sm2, dm2, z_hbm,
             o0, o1, o2, idx_s, idx_d, b0, b1, b2, b3,
             zb, acc, sem_g, sem_s):
        bufs = (b0, b1, b2, b3)
        c = lax.axis_index("c")
        s = lax.axis_index("s")
        row0 = (c * _NS + s) * rw
        pltpu.sync_copy(z_hbm, zb)
        for h_hbm, sm, dm, out in ((h0, sm0, dm0, o0), (h1, sm1, dm1, o1),
                                   (h2, sm2, dm2, o2)):
            _load_idx(sm, dm, idx_s, idx_d, row0, rw)
            _zero_acc(acc, zb, s, rpsz)
            plsc.subcore_barrier()
            _gs_loop(h_hbm, acc, idx_s, idx_d, bufs, sem_g, sem_s, rw)
            plsc.subcore_barrier()
            _writeout(acc, out, c, s, N)
            plsc.subcore_barrier()

    return pl.kernel(
        body,
        out_type=[jax.ShapeDtypeStruct((_NC, N, F), _F32)] * 3,
        mesh=mesh,
        compiler_params=pltpu.CompilerParams(use_tc_tiling_on_sc=False),
        scratch_types=[
            pltpu.VMEM((rw + _PREF, _LANES), jnp.int32),
            pltpu.VMEM((rw, _LANES), jnp.int32),
        ] + [pltpu.VMEM((_LANES, F), _F32)] * _NBUF + [
            pltpu.VMEM((_ZCH, F), _F32),
            pltpu.VMEM_SHARED((nacc, F), _F32),
            pltpu.SemaphoreType.DMA,
            pltpu.SemaphoreType.DMA,
        ],
        name="sc_agg3x%d" % F,
    )


def _bs2(b, f):
    return pl.BlockSpec((b, f), lambda i: (i, 0))


def _bs3(b, f):
    return pl.BlockSpec((_NC, b, f), lambda i: (0, i, 0))


def _bsw(f, fo):
    return pl.BlockSpec((f, fo), lambda i: (0, 0))


def _dinv_of(dref):
    d = dref[...]
    deg = jnp.sum(d[:, :, 0:1], axis=0)
    return 1.0 / jnp.maximum(deg, 1.0)


def _psum(pref, dinv):
    p = pref[...]
    return jnp.sum(p, axis=0) * dinv


def _tc1(p8, degs, x1, x2, w00, w01, w10, w20, N, B=1000):
    nb = N // B

    def body(*refs):
        ps = refs[:8]
        d0, d1, d2, x1r, x2r, w00r, w01r, w10r, w20r = refs[8:17]
        outs = refs[17:]
        i0 = _dinv_of(d0)
        agg = jnp.concatenate([_psum(p, i0) for p in ps], axis=1)
        h256 = jnp.maximum(jnp.dot(agg, w00r[...], precision=_HI), 0.0)
        t0 = jnp.dot(h256, w01r[...], precision=_HI)
        t1 = jnp.dot(x1r[...], w10r[...], precision=_HI)
        t2 = jnp.dot(x2r[...], w20r[...], precision=_HI)
        for v, t in enumerate((t0, t1, t2)):
            outs[2 * v][...] = t[:, :16]
            outs[2 * v + 1][...] = t[:, 16:]

    return pl.pallas_call(
        body,
        grid=(nb,),
        in_specs=([_bs3(B, 16)] * 8 + [_bs3(B, 8)] * 3 + [_bs2(B, 128)] * 2
                  + [_bsw(128, 256), _bsw(256, 32), _bsw(128, 32),
                     _bsw(128, 32)]),
        out_specs=[_bs2(B, 16)] * 6,
        out_shape=[jax.ShapeDtypeStruct((N, 16), _F32)] * 6,
    )(*p8, degs[0], degs[1], degs[2], x1, x2, w00, w01, w10, w20)


def _tc_fuse32(qs, degs, ws, N, B=1000):
    nb = N // B

    def body(*refs):
        q = refs[:6]
        d = refs[6:9]
        w = refs[9:12]
        outs = refs[12:]
        hs = []
        for v in range(3):
            iv = _dinv_of(d[v])
            hs.append(jnp.maximum(
                jnp.concatenate([_psum(q[2 * v], iv),
                                 _psum(q[2 * v + 1], iv)], axis=1), 0.0))
        cc = (hs[0] + hs[1] + hs[2]) / 6.0
        for v in range(3):
            outs[v][...] = jnp.dot(hs[v] * 0.5 + cc, w[v][...], precision=_HI)

    return pl.pallas_call(
        body,
        grid=(nb,),
        in_specs=([_bs3(B, 16)] * 6 + [_bs3(B, 8)] * 3 + [_bsw(32, 16)] * 3),
        out_specs=[_bs2(B, 16)] * 3,
        out_shape=[jax.ShapeDtypeStruct((N, 16), _F32)] * 3,
    )(*qs, degs[0], degs[1], degs[2], ws[0], ws[1], ws[2])


def _tc_fuse(ps, degs, ws, N, fin, B=1000):
    F = ps[0].shape[2]
    Fo = ws[0].shape[1]
    nb = N // B

    def body(p0, p1, p2, d0, d1, d2, wa, wb, wc, u0, u1, u2):
        hs = [jnp.maximum(_psum(p, _dinv_of(d))[:, :fin], 0.0)
              for p, d in ((p0, d0), (p1, d1), (p2, d2))]
        cc = (hs[0] + hs[1] + hs[2]) / 6.0
        for h, wr, u in ((hs[0], wa, u0), (hs[1], wb, u1), (hs[2], wc, u2)):
            u[...] = jnp.dot(h * 0.5 + cc, wr[...], precision=_HI)

    return pl.pallas_call(
        body,
        grid=(nb,),
        in_specs=([_bs3(B, F)] * 3 + [_bs3(B, 8)] * 3 + [_bsw(fin, Fo)] * 3),
        out_specs=[_bs2(B, Fo)] * 3,
        out_shape=[jax.ShapeDtypeStruct((N, Fo), _F32)] * 3,
    )(ps[0], ps[1], ps[2], degs[0], degs[1], degs[2], ws[0], ws[1], ws[2])


def _tc_final(ps, degs, ws, N, B=1000):
    F = ps[0].shape[2]
    Fo = ws[0].shape[1]
    nb = N // B

    def body(p0, p1, p2, d0, d1, d2, wa, wb, wc, o):
        acc = None
        for p, d, wr in ((p0, d0, wa), (p1, d1, wb), (p2, d2, wc)):
            a = _psum(p, _dinv_of(d))[:, :wa.shape[0]]
            t = jnp.dot(a, wr[...], precision=_HI)
            acc = t if acc is None else acc + t
        o[...] = acc / 3.0

    return pl.pallas_call(
        body,
        grid=(nb,),
        in_specs=([_bs3(B, F)] * 3 + [_bs3(B, 8)] * 3 + [_bsw(F, Fo)] * 3),
        out_specs=_bs2(B, Fo),
        out_shape=jax.ShapeDtypeStruct((N, Fo), _F32),
    )(ps[0], ps[1], ps[2], degs[0], degs[1], degs[2], ws[0], ws[1], ws[2])


def _pad_edges(e, ep, n):
    E = e.shape[1]
    src = jnp.concatenate([e[0], jnp.zeros((ep - E,), jnp.int32)])
    dst = jnp.concatenate([e[1], jnp.full((ep - E,), n, jnp.int32)])
    return src.reshape(-1, _LANES), dst.reshape(-1, _LANES)


def kernel(x0, x1, x2, edge_index0, edge_index1, edge_index2,
           W0_0, W0_1, W0_2, W0_3, W0_4, W1_0, W1_1, W1_2, W1_3,
           W2_0, W2_1, W2_2, W2_3):
    N = x0.shape[0]
    E = edge_index0.shape[1]
    rw = 8 * _ceil(E, _NW * _LANES * 8)
    ep = rw * _NW * _LANES

    e0, e1, e2 = [_pad_edges(e, ep, N)
                  for e in (edge_index0, edge_index1, edge_index2)]
    z16 = jnp.zeros((_ZCH, 16), _F32)
    z8 = jnp.zeros((_ZCH, 8), _F32)
    ones_n8 = jnp.ones((N, 8), _F32)
    zeros_n16 = jnp.zeros((N, 16), _F32)
    eye8 = jnp.eye(8, dtype=_F32)

    agg16 = _make_sc_agg3(N, rw, 16)
    agg8 = _make_sc_agg3(N, rw, 8)

    degs = agg8(ones_n8, e0[0], e0[1], ones_n8, e1[0], e1[1],
                ones_n8, e2[0], e2[1], z8)

    xc = [lax.slice_in_dim(x0, 16 * k, 16 * (k + 1), axis=1)
          for k in range(8)] + [zeros_n16]
    p8 = []
    for g in range(3):
        p8.extend(agg16(xc[3 * g], e0[0], e0[1], xc[3 * g + 1], e0[0], e0[1],
                        xc[3 * g + 2], e0[0], e0[1], z16))
    p8 = p8[:8]

    ts = _tc1(p8, degs, x1, x2, W0_0, W0_1, W1_0, W2_0, N)

    q = list(agg16(ts[0], e0[0], e0[1], ts[1], e0[0], e0[1],
                   ts[2], e1[0], e1[1], z16))
    q += list(agg16(ts[3], e1[0], e1[1], ts[4], e2[0], e2[1],
                    ts[5], e2[0], e2[1], z16))
    u = _tc_fuse32(q, degs, (W0_2, W1_1, W2_1), N)

    r = agg16(u[0], e0[0], e0[1], u[1], e1[0], e1[1], u[2], e2[0], e2[1],
              z16)
    v = _tc_fuse(r, degs, (W0_3, W1_2, W2_2), N, fin=16)

    sarr = agg8(v[0], e0[0], e0[1], v[1], e1[0], e1[1], v[2], e2[0], e2[1],
                z8)
    z = _tc_fuse(sarr, degs, (eye8, eye8, eye8), N, fin=8)

    t = agg8(z[0], e0[0], e0[1], z[1], e1[0], e1[1], z[2], e2[0], e2[1], z8)
    return _tc_final(t, degs, (W0_4, W1_3, W2_3), N)


# speedup vs baseline: 1.5808x; 1.5808x over previous
import functools

import jax
import jax.numpy as jnp
from jax import lax
from jax.experimental import pallas as pl
from jax.experimental.pallas import tpu as pltpu
from jax.experimental.pallas import tpu_sc as plsc

_NC = 2
_NS = 16
_NW = _NC * _NS
_LANES = 128
_ZCH = 64
_F32 = jnp.float32
_HI = lax.Precision.HIGHEST


def _ceil(a, b):
    return -(-a // b)


def _zero_acc(acc, zbuf, s, rps):
    base = s * rps
    nf, rem = rps // _ZCH, rps % _ZCH

    def zb(i, _):
        pltpu.sync_copy(zbuf, acc.at[pl.ds(base + i * _ZCH, _ZCH)])
        return _

    lax.fori_loop(0, nf, zb, None)
    if rem:
        pltpu.sync_copy(zbuf.at[pl.ds(0, rem)],
                        acc.at[pl.ds(base + nf * _ZCH, rem)])


_NBUF = 8
_PREF = 4


def _gs_loop(h_hbm, acc, idx_s, idx_d, bufs, sem_g, sem_s, rw):
    def wait_g(slot):
        pltpu.make_async_copy(h_hbm.at[idx_s.at[0]], bufs[slot], sem_g).wait()

    def wait_s():
        pltpu.make_async_copy(bufs[0], acc.at[idx_d.at[0]], sem_s).wait()

    def start_g(j, slot):
        pltpu.async_copy(h_hbm.at[idx_s.at[j]], bufs[slot], sem_g)

    def start_s(j, slot):
        pltpu.async_copy(bufs[slot], acc.at[idx_d.at[j]], sem_s, add=True)

    for j in range(_PREF):
        start_g(j, j)
    for j in range(_NBUF):
        wait_g(j % _NBUF)
        start_s(j, j % _NBUF)
        if j >= _PREF:
            wait_s()
        start_g(j + _PREF, (j + _PREF) % _NBUF)

    def body(g, _):
        j0 = g * _NBUF
        for b in range(_NBUF):
            wait_g(b)
            start_s(j0 + b, b)
            wait_s()
            start_g(j0 + b + _PREF, (b + _PREF) % _NBUF)
        return _

    lax.fori_loop(1, rw // _NBUF, body, None)

    for _e in range(_PREF):
        wait_g(0)
        wait_s()


def _load_idx(srcm, dstm, idx_s, idx_d, row0, rw):
    pltpu.sync_copy(srcm.at[pl.ds(row0, rw)], idx_s.at[pl.ds(0, rw)])
    for r in range(_PREF):
        for k in range(_LANES // 16):
            idx_s[rw + r, pl.ds(16 * k, 16)] = jnp.zeros((16,), jnp.int32)
    pltpu.sync_copy(dstm.at[pl.ds(row0, rw)], idx_d)


def _writeout(acc, out, c, s, n):
    ch = 8 * _ceil(n, _NS * 8)
    last = n - (_NS - 1) * ch

    @pl.when(s < _NS - 1)
    def _():
        pltpu.sync_copy(acc.at[pl.ds(s * ch, ch)],
                        out.at[c, pl.ds(s * ch, ch)])

    @pl.when(s == _NS - 1)
    def _():
        pltpu.sync_copy(acc.at[pl.ds((_NS - 1) * ch, last)],
                        out.at[c, pl.ds((_NS - 1) * ch, last)])


@functools.lru_cache(maxsize=None)
def _make_sc_agg3(N, rw, F):
    nacc = 128 * _ceil(N + 1, 128)
    rpsz = nacc // 16
    mesh = plsc.VectorSubcoreMesh(core_axis_name="c", subcore_axis_name="s",
                                  num_cores=_NC, num_subcores=_NS)

    def body(h0, sm0, dm0, h1, sm1, dm1, h2, sm2, dm2, z_hbm,
             o0, o1, o2, idx_s, idx_d, b0, b1, b2, b3, b4, b5, b6, b7,
             zb, acc, sem_g, sem_s):
        bufs = (b0, b1, b2, b3, b4, b5, b6, b7)
        c = lax.axis_index("c")
        s = lax.axis_index("s")
        row0 = (c * _NS + s) * rw
        pltpu.sync_copy(z_hbm, zb)
        for h_hbm, sm, dm, out in ((h0, sm0, dm0, o0), (h1, sm1, dm1, o1),
                                   (h2, sm2, dm2, o2)):
            _load_idx(sm, dm, idx_s, idx_d, row0, rw)
            _zero_acc(acc, zb, s, rpsz)
            plsc.subcore_barrier()
            _gs_loop(h_hbm, acc, idx_s, idx_d, bufs, sem_g, sem_s, rw)
            plsc.subcore_barrier()
            _writeout(acc, out, c, s, N)
            plsc.subcore_barrier()

    return pl.kernel(
        body,
        out_type=[jax.ShapeDtypeStruct((_NC, N, F), _F32)] * 3,
        mesh=mesh,
        compiler_params=pltpu.CompilerParams(use_tc_tiling_on_sc=False),
        scratch_types=[
            pltpu.VMEM((rw + _PREF, _LANES), jnp.int32),
            pltpu.VMEM((rw, _LANES), jnp.int32),
        ] + [pltpu.VMEM((_LANES, F), _F32)] * _NBUF + [
            pltpu.VMEM((_ZCH, F), _F32),
            pltpu.VMEM_SHARED((nacc, F), _F32),
            pltpu.SemaphoreType.DMA,
            pltpu.SemaphoreType.DMA,
        ],
        name="sc_agg3x%d" % F,
    )


def _bs2(b, f):
    return pl.BlockSpec((b, f), lambda i: (i, 0))


def _bs3(b, f):
    return pl.BlockSpec((_NC, b, f), lambda i: (0, i, 0))


def _bsw(f, fo):
    return pl.BlockSpec((f, fo), lambda i: (0, 0))


def _dinv_of(dref):
    d = dref[...]
    deg = jnp.sum(d[:, :, 0:1], axis=0)
    return 1.0 / jnp.maximum(deg, 1.0)


def _psum(pref, dinv):
    p = pref[...]
    return jnp.sum(p, axis=0) * dinv


def _tc1(p8, degs, x1, x2, w00, w01, w10, w20, N, B=1000):
    nb = N // B

    def body(*refs):
        ps = refs[:8]
        d0, d1, d2, x1r, x2r, w00r, w01r, w10r, w20r = refs[8:17]
        outs = refs[17:]
        i0 = _dinv_of(d0)
        agg = jnp.concatenate([_psum(p, i0) for p in ps], axis=1)
        h256 = jnp.maximum(jnp.dot(agg, w00r[...], precision=_HI), 0.0)
        t0 = jnp.dot(h256, w01r[...], precision=_HI)
        t1 = jnp.dot(x1r[...], w10r[...], precision=_HI)
        t2 = jnp.dot(x2r[...], w20r[...], precision=_HI)
        for v, t in enumerate((t0, t1, t2)):
            outs[2 * v][...] = t[:, :16]
            outs[2 * v + 1][...] = t[:, 16:]

    return pl.pallas_call(
        body,
        grid=(nb,),
        in_specs=([_bs3(B, 16)] * 8 + [_bs3(B, 8)] * 3 + [_bs2(B, 128)] * 2
                  + [_bsw(128, 256), _bsw(256, 32), _bsw(128, 32),
                     _bsw(128, 32)]),
        out_specs=[_bs2(B, 16)] * 6,
        out_shape=[jax.ShapeDtypeStruct((N, 16), _F32)] * 6,
    )(*p8, degs[0], degs[1], degs[2], x1, x2, w00, w01, w10, w20)


def _tc_fuse32(qs, degs, ws, N, B=1000):
    nb = N // B

    def body(*refs):
        q = refs[:6]
        d = refs[6:9]
        w = refs[9:12]
        outs = refs[12:]
        hs = []
        for v in range(3):
            iv = _dinv_of(d[v])
            hs.append(jnp.maximum(
                jnp.concatenate([_psum(q[2 * v], iv),
                                 _psum(q[2 * v + 1], iv)], axis=1), 0.0))
        cc = (hs[0] + hs[1] + hs[2]) / 6.0
        for v in range(3):
            outs[v][...] = jnp.dot(hs[v] * 0.5 + cc, w[v][...], precision=_HI)

    return pl.pallas_call(
        body,
        grid=(nb,),
        in_specs=([_bs3(B, 16)] * 6 + [_bs3(B, 8)] * 3 + [_bsw(32, 16)] * 3),
        out_specs=[_bs2(B, 16)] * 3,
        out_shape=[jax.ShapeDtypeStruct((N, 16), _F32)] * 3,
    )(*qs, degs[0], degs[1], degs[2], ws[0], ws[1], ws[2])


def _tc_fuse(ps, degs, ws, N, fin, B=1000):
    F = ps[0].shape[2]
    Fo = ws[0].shape[1]
    nb = N // B

    def body(p0, p1, p2, d0, d1, d2, wa, wb, wc, u0, u1, u2):
        hs = [jnp.maximum(_psum(p, _dinv_of(d))[:, :fin], 0.0)
              for p, d in ((p0, d0), (p1, d1), (p2, d2))]
        cc = (hs[0] + hs[1] + hs[2]) / 6.0
        for h, wr, u in ((hs[0], wa, u0), (hs[1], wb, u1), (hs[2], wc, u2)):
            u[...] = jnp.dot(h * 0.5 + cc, wr[...], precision=_HI)

    return pl.pallas_call(
        body,
        grid=(nb,),
        in_specs=([_bs3(B, F)] * 3 + [_bs3(B, 8)] * 3 + [_bsw(fin, Fo)] * 3),
        out_specs=[_bs2(B, Fo)] * 3,
        out_shape=[jax.ShapeDtypeStruct((N, Fo), _F32)] * 3,
    )(ps[0], ps[1], ps[2], degs[0], degs[1], degs[2], ws[0], ws[1], ws[2])


def _tc_final(ps, degs, ws, N, B=1000):
    F = ps[0].shape[2]
    Fo = ws[0].shape[1]
    nb = N // B

    def body(p0, p1, p2, d0, d1, d2, wa, wb, wc, o):
        acc = None
        for p, d, wr in ((p0, d0, wa), (p1, d1, wb), (p2, d2, wc)):
            a = _psum(p, _dinv_of(d))[:, :wa.shape[0]]
            t = jnp.dot(a, wr[...], precision=_HI)
            acc = t if acc is None else acc + t
        o[...] = acc / 3.0

    return pl.pallas_call(
        body,
        grid=(nb,),
        in_specs=([_bs3(B, F)] * 3 + [_bs3(B, 8)] * 3 + [_bsw(F, Fo)] * 3),
        out_specs=_bs2(B, Fo),
        out_shape=jax.ShapeDtypeStruct((N, Fo), _F32),
    )(ps[0], ps[1], ps[2], degs[0], degs[1], degs[2], ws[0], ws[1], ws[2])


def _pad_edges(e, ep, n):
    E = e.shape[1]
    src = jnp.concatenate([e[0], jnp.zeros((ep - E,), jnp.int32)])
    dst = jnp.concatenate([e[1], jnp.full((ep - E,), n, jnp.int32)])
    return src.reshape(-1, _LANES), dst.reshape(-1, _LANES)


def kernel(x0, x1, x2, edge_index0, edge_index1, edge_index2,
           W0_0, W0_1, W0_2, W0_3, W0_4, W1_0, W1_1, W1_2, W1_3,
           W2_0, W2_1, W2_2, W2_3):
    N = x0.shape[0]
    E = edge_index0.shape[1]
    rw = 8 * _ceil(E, _NW * _LANES * 8)
    ep = rw * _NW * _LANES

    e0, e1, e2 = [_pad_edges(e, ep, N)
                  for e in (edge_index0, edge_index1, edge_index2)]
    z16 = jnp.zeros((_ZCH, 16), _F32)
    z8 = jnp.zeros((_ZCH, 8), _F32)
    ones_n8 = jnp.ones((N, 8), _F32)
    zeros_n16 = jnp.zeros((N, 16), _F32)
    eye8 = jnp.eye(8, dtype=_F32)

    agg16 = _make_sc_agg3(N, rw, 16)
    agg8 = _make_sc_agg3(N, rw, 8)

    degs = agg8(ones_n8, e0[0], e0[1], ones_n8, e1[0], e1[1],
                ones_n8, e2[0], e2[1], z8)

    xc = [lax.slice_in_dim(x0, 16 * k, 16 * (k + 1), axis=1)
          for k in range(8)] + [zeros_n16]
    p8 = []
    for g in range(3):
        p8.extend(agg16(xc[3 * g], e0[0], e0[1], xc[3 * g + 1], e0[0], e0[1],
                        xc[3 * g + 2], e0[0], e0[1], z16))
    p8 = p8[:8]

    ts = _tc1(p8, degs, x1, x2, W0_0, W0_1, W1_0, W2_0, N)

    q = list(agg16(ts[0], e0[0], e0[1], ts[1], e0[0], e0[1],
                   ts[2], e1[0], e1[1], z16))
    q += list(agg16(ts[3], e1[0], e1[1], ts[4], e2[0], e2[1],
                    ts[5], e2[0], e2[1], z16))
    u = _tc_fuse32(q, degs, (W0_2, W1_1, W2_1), N)

    r = agg16(u[0], e0[0], e0[1], u[1], e1[0], e1[1], u[2], e2[0], e2[1],
              z16)
    v = _tc_fuse(r, degs, (W0_3, W1_2, W2_2), N, fin=16)

    sarr = agg8(v[0], e0[0], e0[1], v[1], e1[0], e1[1], v[2], e2[0], e2[1],
                z8)
    z = _tc_fuse(sarr, degs, (eye8, eye8, eye8), N, fin=8)

    t = agg8(z[0], e0[0], e0[1], z[1], e1[0], e1[1], z[2], e2[0], e2[1], z8)
    return _tc_final(t, degs, (W0_4, W1_3, W2_3), N)


# speedup vs baseline: 1.7416x; 1.1017x over previous
import functools

import jax
import jax.numpy as jnp
from jax import lax
from jax.experimental import pallas as pl
from jax.experimental.pallas import tpu as pltpu
from jax.experimental.pallas import tpu_sc as plsc

_NC = 2
_NS = 16
_NW = _NC * _NS
_LANES = 128
_ZCH = 64
_F32 = jnp.float32
_HI = lax.Precision.HIGHEST


def _ceil(a, b):
    return -(-a // b)


def _zero_acc(acc, zbuf, s, rps):
    base = s * rps
    nf, rem = rps // _ZCH, rps % _ZCH

    def zb(i, _):
        pltpu.sync_copy(zbuf, acc.at[pl.ds(base + i * _ZCH, _ZCH)])
        return _

    lax.fori_loop(0, nf, zb, None)
    if rem:
        pltpu.sync_copy(zbuf.at[pl.ds(0, rem)],
                        acc.at[pl.ds(base + nf * _ZCH, rem)])


_NBUF = 8
_PREF = 4


def _gs_loop(h_hbm, acc, idx_s, idx_d, bufs, sem_g, sem_s, rw):
    def wait_g(slot):
        pltpu.make_async_copy(h_hbm.at[idx_s.at[0]], bufs[slot], sem_g).wait()

    def wait_s():
        pltpu.make_async_copy(bufs[0], acc.at[idx_d.at[0]], sem_s).wait()

    def start_g(j, slot):
        pltpu.async_copy(h_hbm.at[idx_s.at[j]], bufs[slot], sem_g)

    def start_s(j, slot):
        pltpu.async_copy(bufs[slot], acc.at[idx_d.at[j]], sem_s, add=True)

    for j in range(_PREF):
        start_g(j, j)
    for j in range(_NBUF):
        wait_g(j % _NBUF)
        start_s(j, j % _NBUF)
        if j >= _PREF:
            wait_s()
        start_g(j + _PREF, (j + _PREF) % _NBUF)

    def body(g, _):
        j0 = g * _NBUF
        for b in range(_NBUF):
            wait_g(b)
            start_s(j0 + b, b)
            wait_s()
            start_g(j0 + b + _PREF, (b + _PREF) % _NBUF)
        return _

    lax.fori_loop(1, rw // _NBUF, body, None)

    for _e in range(_PREF):
        wait_g(0)
        wait_s()


def _load_idx(srcm, dstm, idx_s, idx_d, row0, rw):
    pltpu.sync_copy(srcm.at[pl.ds(row0, rw)], idx_s.at[pl.ds(0, rw)])
    for r in range(_PREF):
        for k in range(_LANES // 16):
            idx_s[rw + r, pl.ds(16 * k, 16)] = jnp.zeros((16,), jnp.int32)
    pltpu.sync_copy(dstm.at[pl.ds(row0, rw)], idx_d)


def _writeout(acc, out, c, s, n):
    ch = 8 * _ceil(n, _NS * 8)
    last = n - (_NS - 1) * ch

    @pl.when(s < _NS - 1)
    def _():
        pltpu.sync_copy(acc.at[pl.ds(s * ch, ch)],
                        out.at[c, pl.ds(s * ch, ch)])

    @pl.when(s == _NS - 1)
    def _():
        pltpu.sync_copy(acc.at[pl.ds((_NS - 1) * ch, last)],
                        out.at[c, pl.ds((_NS - 1) * ch, last)])


@functools.lru_cache(maxsize=None)
def _make_sc_agg(N, rw, F):
    nacc = 128 * _ceil(N + 1, 128)
    rpsz = nacc // 16
    mesh = plsc.VectorSubcoreMesh(core_axis_name="c", subcore_axis_name="s",
                                  num_cores=_NC, num_subcores=_NS)

    def body(h_hbm, srcm, dstm, z_hbm, out, idx_s, idx_d,
             b0, b1, b2, b3, b4, b5, b6, b7, zb, acc, sem_g, sem_s):
        bufs = (b0, b1, b2, b3, b4, b5, b6, b7)
        c = lax.axis_index("c")
        s = lax.axis_index("s")
        row0 = (c * _NS + s) * rw
        pltpu.sync_copy(z_hbm, zb)
        _load_idx(srcm, dstm, idx_s, idx_d, row0, rw)
        _zero_acc(acc, zb, s, rpsz)
        plsc.subcore_barrier()
        _gs_loop(h_hbm, acc, idx_s, idx_d, bufs, sem_g, sem_s, rw)
        plsc.subcore_barrier()
        _writeout(acc, out, c, s, N)

    return pl.kernel(
        body,
        out_type=jax.ShapeDtypeStruct((_NC, N, F), _F32),
        mesh=mesh,
        compiler_params=pltpu.CompilerParams(use_tc_tiling_on_sc=False),
        scratch_types=[
            pltpu.VMEM((rw + _PREF, _LANES), jnp.int32),
            pltpu.VMEM((rw, _LANES), jnp.int32),
        ] + [pltpu.VMEM((_LANES, F), _F32)] * _NBUF + [
            pltpu.VMEM((_ZCH, F), _F32),
            pltpu.VMEM_SHARED((nacc, F), _F32),
            pltpu.SemaphoreType.DMA,
            pltpu.SemaphoreType.DMA,
        ],
        name="sc_agg%d" % F,
    )


@functools.lru_cache(maxsize=None)
def _make_sc_deg(N, rw):
    nacc = 128 * _ceil(N + 1, 128)
    rpsz = nacc // 16
    lag = 8
    mesh = plsc.VectorSubcoreMesh(core_axis_name="c", subcore_axis_name="s",
                                  num_cores=_NC, num_subcores=_NS)

    def body(dstm, ones_hbm, z_hbm, out, idx_d, onesb, zb, acc, sem_s):
        c = lax.axis_index("c")
        s = lax.axis_index("s")
        row0 = (c * _NS + s) * rw
        pltpu.sync_copy(z_hbm, zb)
        pltpu.sync_copy(ones_hbm, onesb)
        pltpu.sync_copy(dstm.at[pl.ds(row0, rw)], idx_d)
        _zero_acc(acc, zb, s, rpsz)
        plsc.subcore_barrier()
        for j in range(lag):
            pltpu.async_copy(onesb, acc.at[idx_d.at[j]], sem_s, add=True)

        def step(j, _):
            pltpu.async_copy(onesb, acc.at[idx_d.at[j]], sem_s, add=True)
            pltpu.make_async_copy(onesb, acc.at[idx_d.at[0]], sem_s).wait()
            return _

        lax.fori_loop(lag, rw, step, None)
        for _e in range(lag):
            pltpu.make_async_copy(onesb, acc.at[idx_d.at[0]], sem_s).wait()
        plsc.subcore_barrier()
        _writeout(acc, out, c, s, N)

    return pl.kernel(
        body,
        out_type=jax.ShapeDtypeStruct((_NC, N, 8), _F32),
        mesh=mesh,
        compiler_params=pltpu.CompilerParams(use_tc_tiling_on_sc=False),
        scratch_types=[
            pltpu.VMEM((rw, _LANES), jnp.int32),
            pltpu.VMEM((_LANES, 8), _F32),
            pltpu.VMEM((_ZCH, 8), _F32),
            pltpu.VMEM_SHARED((nacc, 8), _F32),
            pltpu.SemaphoreType.DMA,
        ],
        name="sc_deg",
    )


def _bs2(b, f):
    return pl.BlockSpec((b, f), lambda i: (i, 0))


def _bs3(b, f):
    return pl.BlockSpec((_NC, b, f), lambda i: (0, i, 0))


def _bsw(f, fo):
    return pl.BlockSpec((f, fo), lambda i: (0, 0))


def _dinv_of(dref):
    d = dref[...]
    deg = jnp.sum(d[:, :, 0:1], axis=0)
    return 1.0 / jnp.maximum(deg, 1.0)


def _psum(pref, dinv):
    p = pref[...]
    return jnp.sum(p, axis=0) * dinv


def _tc1(p8, degs, x1, x2, w00, w01, w10, w20, N, B=1000):
    nb = N // B

    def body(*refs):
        ps = refs[:8]
        d0, d1, d2, x1r, x2r, w00r, w01r, w10r, w20r = refs[8:17]
        outs = refs[17:]
        i0 = _dinv_of(d0)
        agg = jnp.concatenate([_psum(p, i0) for p in ps], axis=1)
        h256 = jnp.maximum(jnp.dot(agg, w00r[...], precision=_HI), 0.0)
        t0 = jnp.dot(h256, w01r[...], precision=_HI)
        t1 = jnp.dot(x1r[...], w10r[...], precision=_HI)
        t2 = jnp.dot(x2r[...], w20r[...], precision=_HI)
        for v, t in enumerate((t0, t1, t2)):
            outs[2 * v][...] = t[:, :16]
            outs[2 * v + 1][...] = t[:, 16:]

    return pl.pallas_call(
        body,
        grid=(nb,),
        in_specs=([_bs3(B, 16)] * 8 + [_bs3(B, 8)] * 3 + [_bs2(B, 128)] * 2
                  + [_bsw(128, 256), _bsw(256, 32), _bsw(128, 32),
                     _bsw(128, 32)]),
        out_specs=[_bs2(B, 16)] * 6,
        out_shape=[jax.ShapeDtypeStruct((N, 16), _F32)] * 6,
    )(*p8, degs[0], degs[1], degs[2], x1, x2, w00, w01, w10, w20)


def _tc_fuse32(qs, degs, ws, N, B=1000):
    nb = N // B

    def body(*refs):
        q = refs[:6]
        d = refs[6:9]
        w = refs[9:12]
        outs = refs[12:]
        hs = []
        for v in range(3):
            iv = _dinv_of(d[v])
            hs.append(jnp.maximum(
                jnp.concatenate([_psum(q[2 * v], iv),
                                 _psum(q[2 * v + 1], iv)], axis=1), 0.0))
        cc = (hs[0] + hs[1] + hs[2]) / 6.0
        for v in range(3):
            outs[v][...] = jnp.dot(hs[v] * 0.5 + cc, w[v][...], precision=_HI)

    return pl.pallas_call(
        body,
        grid=(nb,),
        in_specs=([_bs3(B, 16)] * 6 + [_bs3(B, 8)] * 3 + [_bsw(32, 16)] * 3),
        out_specs=[_bs2(B, 16)] * 3,
        out_shape=[jax.ShapeDtypeStruct((N, 16), _F32)] * 3,
    )(*qs, degs[0], degs[1], degs[2], ws[0], ws[1], ws[2])


def _tc_fuse(ps, degs, ws, N, fin, B=1000):
    F = ps[0].shape[2]
    Fo = ws[0].shape[1]
    nb = N // B

    def body(p0, p1, p2, d0, d1, d2, wa, wb, wc, u0, u1, u2):
        hs = [jnp.maximum(_psum(p, _dinv_of(d))[:, :fin], 0.0)
              for p, d in ((p0, d0), (p1, d1), (p2, d2))]
        cc = (hs[0] + hs[1] + hs[2]) / 6.0
        for h, wr, u in ((hs[0], wa, u0), (hs[1], wb, u1), (hs[2], wc, u2)):
            u[...] = jnp.dot(h * 0.5 + cc, wr[...], precision=_HI)

    return pl.pallas_call(
        body,
        grid=(nb,),
        in_specs=([_bs3(B, F)] * 3 + [_bs3(B, 8)] * 3 + [_bsw(fin, Fo)] * 3),
        out_specs=[_bs2(B, Fo)] * 3,
        out_shape=[jax.ShapeDtypeStruct((N, Fo), _F32)] * 3,
    )(ps[0], ps[1], ps[2], degs[0], degs[1], degs[2], ws[0], ws[1], ws[2])


def _tc_final(ps, degs, ws, N, B=1000):
    F = ps[0].shape[2]
    Fo = ws[0].shape[1]
    nb = N // B

    def body(p0, p1, p2, d0, d1, d2, wa, wb, wc, o):
        acc = None
        for p, d, wr in ((p0, d0, wa), (p1, d1, wb), (p2, d2, wc)):
            a = _psum(p, _dinv_of(d))[:, :wa.shape[0]]
            t = jnp.dot(a, wr[...], precision=_HI)
            acc = t if acc is None else acc + t
        o[...] = acc / 3.0

    return pl.pallas_call(
        body,
        grid=(nb,),
        in_specs=([_bs3(B, F)] * 3 + [_bs3(B, 8)] * 3 + [_bsw(F, Fo)] * 3),
        out_specs=_bs2(B, Fo),
        out_shape=jax.ShapeDtypeStruct((N, Fo), _F32),
    )(ps[0], ps[1], ps[2], degs[0], degs[1], degs[2], ws[0], ws[1], ws[2])


def _pad_edges(e, ep, n):
    E = e.shape[1]
    src = jnp.concatenate([e[0], jnp.zeros((ep - E,), jnp.int32)])
    dst = jnp.concatenate([e[1], jnp.full((ep - E,), n, jnp.int32)])
    return src.reshape(-1, _LANES), dst.reshape(-1, _LANES)


def kernel(x0, x1, x2, edge_index0, edge_index1, edge_index2,
           W0_0, W0_1, W0_2, W0_3, W0_4, W1_0, W1_1, W1_2, W1_3,
           W2_0, W2_1, W2_2, W2_3):
    N = x0.shape[0]
    E = edge_index0.shape[1]
    rw = 8 * _ceil(E, _NW * _LANES * 8)
    ep = rw * _NW * _LANES

    e0, e1, e2 = [_pad_edges(e, ep, N)
                  for e in (edge_index0, edge_index1, edge_index2)]
    z16 = jnp.zeros((_ZCH, 16), _F32)
    z8 = jnp.zeros((_ZCH, 8), _F32)
    eye8 = jnp.eye(8, dtype=_F32)

    agg16 = _make_sc_agg(N, rw, 16)
    agg8 = _make_sc_agg(N, rw, 8)
    deg = _make_sc_deg(N, rw)

    ones_t = jnp.ones((_LANES, 8), _F32)
    degs = [deg(dm, ones_t, z8) for _, dm in (e0, e1, e2)]

    p8 = [agg16(lax.slice_in_dim(x0, 16 * k, 16 * (k + 1), axis=1),
                e0[0], e0[1], z16) for k in range(8)]

    ts = _tc1(p8, degs, x1, x2, W0_0, W0_1, W1_0, W2_0, N)

    ev = (e0, e0, e1, e1, e2, e2)
    q = [agg16(ts[i], ev[i][0], ev[i][1], z16) for i in range(6)]
    u = _tc_fuse32(q, degs, (W0_2, W1_1, W2_1), N)

    r = [agg16(u[i], (e0, e1, e2)[i][0], (e0, e1, e2)[i][1], z16)
         for i in range(3)]
    v = _tc_fuse(r, degs, (W0_3, W1_2, W2_2), N, fin=16)

    sarr = [agg8(v[i], (e0, e1, e2)[i][0], (e0, e1, e2)[i][1], z8)
            for i in range(3)]
    z = _tc_fuse(sarr, degs, (eye8, eye8, eye8), N, fin=8)

    t = [agg8(z[i], (e0, e1, e2)[i][0], (e0, e1, e2)[i][1], z8)
         for i in range(3)]
    return _tc_final(t, degs, (W0_4, W1_3, W2_3), N)


# speedup vs baseline: 1.7528x; 1.0064x over previous
import functools

import jax
import jax.numpy as jnp
from jax import lax
from jax.experimental import pallas as pl
from jax.experimental.pallas import tpu as pltpu
from jax.experimental.pallas import tpu_sc as plsc

_NC = 2
_NS = 16
_NW = _NC * _NS
_LANES = 128
_ZCH = 64
_F32 = jnp.float32
_HI = lax.Precision.HIGHEST


def _ceil(a, b):
    return -(-a // b)


def _zero_acc(acc, zbuf, s, rps, sem):
    base = s * rps
    nf, rem = rps // _ZCH, rps % _ZCH

    def zb(i, _):
        pltpu.async_copy(zbuf, acc.at[pl.ds(base + i * _ZCH, _ZCH)], sem)
        return _

    lax.fori_loop(0, nf, zb, None)
    if rem:
        pltpu.async_copy(zbuf.at[pl.ds(0, rem)],
                         acc.at[pl.ds(base + nf * _ZCH, rem)], sem)

    def dr(i, _):
        pltpu.make_async_copy(zbuf, acc.at[pl.ds(base, _ZCH)], sem).wait()
        return _

    lax.fori_loop(0, nf, dr, None)
    if rem:
        pltpu.make_async_copy(zbuf.at[pl.ds(0, rem)],
                              acc.at[pl.ds(base, rem)], sem).wait()


_NBUF = 8
_PREF = 4


def _gs_loop(h_hbm, acc, idx_s, idx_d, bufs, sem_g, sem_s, rw):
    def wait_g(slot):
        pltpu.make_async_copy(h_hbm.at[idx_s.at[0]], bufs[slot], sem_g).wait()

    def wait_s():
        pltpu.make_async_copy(bufs[0], acc.at[idx_d.at[0]], sem_s).wait()

    def start_g(j, slot):
        pltpu.async_copy(h_hbm.at[idx_s.at[j]], bufs[slot], sem_g)

    def start_s(j, slot):
        pltpu.async_copy(bufs[slot], acc.at[idx_d.at[j]], sem_s, add=True)

    for j in range(_PREF):
        start_g(j, j)
    for j in range(_NBUF):
        wait_g(j % _NBUF)
        start_s(j, j % _NBUF)
        if j >= _PREF:
            wait_s()
        start_g(j + _PREF, (j + _PREF) % _NBUF)

    def body(g, _):
        j0 = g * _NBUF
        for b in range(_NBUF):
            wait_g(b)
            start_s(j0 + b, b)
            wait_s()
            start_g(j0 + b + _PREF, (b + _PREF) % _NBUF)
        return _

    lax.fori_loop(1, rw // _NBUF, body, None)

    for _e in range(_PREF):
        wait_g(0)
        wait_s()


def _load_idx(srcm, dstm, idx_s, idx_d, row0, rw):
    pltpu.sync_copy(srcm.at[pl.ds(row0, rw)], idx_s.at[pl.ds(0, rw)])
    for r in range(_PREF):
        for k in range(_LANES // 16):
            idx_s[rw + r, pl.ds(16 * k, 16)] = jnp.zeros((16,), jnp.int32)
    pltpu.sync_copy(dstm.at[pl.ds(row0, rw)], idx_d)


def _writeout(acc, out, c, s, n):
    ch = 8 * _ceil(n, _NS * 8)
    last = n - (_NS - 1) * ch

    @pl.when(s < _NS - 1)
    def _():
        pltpu.sync_copy(acc.at[pl.ds(s * ch, ch)],
                        out.at[c, pl.ds(s * ch, ch)])

    @pl.when(s == _NS - 1)
    def _():
        pltpu.sync_copy(acc.at[pl.ds((_NS - 1) * ch, last)],
                        out.at[c, pl.ds((_NS - 1) * ch, last)])


@functools.lru_cache(maxsize=None)
def _make_sc_agg(N, rw, F):
    nacc = 128 * _ceil(N + 1, 128)
    rpsz = nacc // 16
    mesh = plsc.VectorSubcoreMesh(core_axis_name="c", subcore_axis_name="s",
                                  num_cores=_NC, num_subcores=_NS)

    def body(h_hbm, srcm, dstm, z_hbm, out, idx_s, idx_d,
             b0, b1, b2, b3, b4, b5, b6, b7, zb, acc, sem_g, sem_s):
        bufs = (b0, b1, b2, b3, b4, b5, b6, b7)
        c = lax.axis_index("c")
        s = lax.axis_index("s")
        row0 = (c * _NS + s) * rw
        pltpu.sync_copy(z_hbm, zb)
        _load_idx(srcm, dstm, idx_s, idx_d, row0, rw)
        _zero_acc(acc, zb, s, rpsz, sem_s)
        plsc.subcore_barrier()
        _gs_loop(h_hbm, acc, idx_s, idx_d, bufs, sem_g, sem_s, rw)
        plsc.subcore_barrier()
        _writeout(acc, out, c, s, N)

    return pl.kernel(
        body,
        out_type=jax.ShapeDtypeStruct((_NC, N, F), _F32),
        mesh=mesh,
        compiler_params=pltpu.CompilerParams(use_tc_tiling_on_sc=False),
        scratch_types=[
            pltpu.VMEM((rw + _PREF, _LANES), jnp.int32),
            pltpu.VMEM((rw, _LANES), jnp.int32),
        ] + [pltpu.VMEM((_LANES, F), _F32)] * _NBUF + [
            pltpu.VMEM((_ZCH, F), _F32),
            pltpu.VMEM_SHARED((nacc, F), _F32),
            pltpu.SemaphoreType.DMA,
            pltpu.SemaphoreType.DMA,
        ],
        name="sc_agg%d" % F,
    )


@functools.lru_cache(maxsize=None)
def _make_sc_deg(N, rw):
    nacc = 128 * _ceil(N + 1, 128)
    rpsz = nacc // 16
    lag = 8
    mesh = plsc.VectorSubcoreMesh(core_axis_name="c", subcore_axis_name="s",
                                  num_cores=_NC, num_subcores=_NS)

    def body(dstm, ones_hbm, z_hbm, out, idx_d, onesb, zb, acc, sem_s):
        c = lax.axis_index("c")
        s = lax.axis_index("s")
        row0 = (c * _NS + s) * rw
        pltpu.sync_copy(z_hbm, zb)
        pltpu.sync_copy(ones_hbm, onesb)
        pltpu.sync_copy(dstm.at[pl.ds(row0, rw)], idx_d)
        _zero_acc(acc, zb, s, rpsz, sem_s)
        plsc.subcore_barrier()
        for j in range(lag):
            pltpu.async_copy(onesb, acc.at[idx_d.at[j]], sem_s, add=True)

        def step(j, _):
            pltpu.async_copy(onesb, acc.at[idx_d.at[j]], sem_s, add=True)
            pltpu.make_async_copy(onesb, acc.at[idx_d.at[0]], sem_s).wait()
            return _

        lax.fori_loop(lag, rw, step, None)
        for _e in range(lag):
            pltpu.make_async_copy(onesb, acc.at[idx_d.at[0]], sem_s).wait()
        plsc.subcore_barrier()
        _writeout(acc, out, c, s, N)

    return pl.kernel(
        body,
        out_type=jax.ShapeDtypeStruct((_NC, N, 8), _F32),
        mesh=mesh,
        compiler_params=pltpu.CompilerParams(use_tc_tiling_on_sc=False),
        scratch_types=[
            pltpu.VMEM((rw, _LANES), jnp.int32),
            pltpu.VMEM((_LANES, 8), _F32),
            pltpu.VMEM((_ZCH, 8), _F32),
            pltpu.VMEM_SHARED((nacc, 8), _F32),
            pltpu.SemaphoreType.DMA,
        ],
        name="sc_deg",
    )


def _bs2(b, f):
    return pl.BlockSpec((b, f), lambda i: (i, 0))


def _bs3(b, f):
    return pl.BlockSpec((_NC, b, f), lambda i: (0, i, 0))


def _bsw(f, fo):
    return pl.BlockSpec((f, fo), lambda i: (0, 0))


def _dinv_of(dref):
    d = dref[...]
    deg = jnp.sum(d[:, :, 0:1], axis=0)
    return 1.0 / jnp.maximum(deg, 1.0)


def _psum(pref, dinv):
    p = pref[...]
    return jnp.sum(p, axis=0) * dinv


def _tc1(p8, degs, x1, x2, w00, w01, w10, w20, N, B=1000):
    nb = N // B

    def body(*refs):
        ps = refs[:8]
        d0, d1, d2, x1r, x2r, w00r, w01r, w10r, w20r = refs[8:17]
        outs = refs[17:]
        i0 = _dinv_of(d0)
        agg = jnp.concatenate([_psum(p, i0) for p in ps], axis=1)
        h256 = jnp.maximum(jnp.dot(agg, w00r[...], precision=_HI), 0.0)
        t0 = jnp.dot(h256, w01r[...], precision=_HI)
        t1 = jnp.dot(x1r[...], w10r[...], precision=_HI)
        t2 = jnp.dot(x2r[...], w20r[...], precision=_HI)
        for v, t in enumerate((t0, t1, t2)):
            outs[2 * v][...] = t[:, :16]
            outs[2 * v + 1][...] = t[:, 16:]

    return pl.pallas_call(
        body,
        grid=(nb,),
        in_specs=([_bs3(B, 16)] * 8 + [_bs3(B, 8)] * 3 + [_bs2(B, 128)] * 2
                  + [_bsw(128, 256), _bsw(256, 32), _bsw(128, 32),
                     _bsw(128, 32)]),
        out_specs=[_bs2(B, 16)] * 6,
        out_shape=[jax.ShapeDtypeStruct((N, 16), _F32)] * 6,
    )(*p8, degs[0], degs[1], degs[2], x1, x2, w00, w01, w10, w20)


def _tc_fuse32(qs, degs, ws, N, B=1000):
    nb = N // B

    def body(*refs):
        q = refs[:6]
        d = refs[6:9]
        w = refs[9:12]
        outs = refs[12:]
        hs = []
        for v in range(3):
            iv = _dinv_of(d[v])
            hs.append(jnp.maximum(
                jnp.concatenate([_psum(q[2 * v], iv),
                                 _psum(q[2 * v + 1], iv)], axis=1), 0.0))
        cc = (hs[0] + hs[1] + hs[2]) / 6.0
        for v in range(3):
            outs[v][...] = jnp.dot(hs[v] * 0.5 + cc, w[v][...], precision=_HI)

    return pl.pallas_call(
        body,
        grid=(nb,),
        in_specs=([_bs3(B, 16)] * 6 + [_bs3(B, 8)] * 3 + [_bsw(32, 16)] * 3),
        out_specs=[_bs2(B, 16)] * 3,
        out_shape=[jax.ShapeDtypeStruct((N, 16), _F32)] * 3,
    )(*qs, degs[0], degs[1], degs[2], ws[0], ws[1], ws[2])


def _tc_fuse(ps, degs, ws, N, fin, B=1000):
    F = ps[0].shape[2]
    Fo = ws[0].shape[1]
    nb = N // B

    def body(p0, p1, p2, d0, d1, d2, wa, wb, wc, u0, u1, u2):
        hs = [jnp.maximum(_psum(p, _dinv_of(d))[:, :fin], 0.0)
              for p, d in ((p0, d0), (p1, d1), (p2, d2))]
        cc = (hs[0] + hs[1] + hs[2]) / 6.0
        for h, wr, u in ((hs[0], wa, u0), (hs[1], wb, u1), (hs[2], wc, u2)):
            u[...] = jnp.dot(h * 0.5 + cc, wr[...], precision=_HI)

    return pl.pallas_call(
        body,
        grid=(nb,),
        in_specs=([_bs3(B, F)] * 3 + [_bs3(B, 8)] * 3 + [_bsw(fin, Fo)] * 3),
        out_specs=[_bs2(B, Fo)] * 3,
        out_shape=[jax.ShapeDtypeStruct((N, Fo), _F32)] * 3,
    )(ps[0], ps[1], ps[2], degs[0], degs[1], degs[2], ws[0], ws[1], ws[2])


def _tc_final(ps, degs, ws, N, B=1000):
    F = ps[0].shape[2]
    Fo = ws[0].shape[1]
    nb = N // B

    def body(p0, p1, p2, d0, d1, d2, wa, wb, wc, o):
        acc = None
        for p, d, wr in ((p0, d0, wa), (p1, d1, wb), (p2, d2, wc)):
            a = _psum(p, _dinv_of(d))[:, :wa.shape[0]]
            t = jnp.dot(a, wr[...], precision=_HI)
            acc = t if acc is None else acc + t
        o[...] = acc / 3.0

    return pl.pallas_call(
        body,
        grid=(nb,),
        in_specs=([_bs3(B, F)] * 3 + [_bs3(B, 8)] * 3 + [_bsw(F, Fo)] * 3),
        out_specs=_bs2(B, Fo),
        out_shape=jax.ShapeDtypeStruct((N, Fo), _F32),
    )(ps[0], ps[1], ps[2], degs[0], degs[1], degs[2], ws[0], ws[1], ws[2])


def _pad_edges(e, ep, n):
    E = e.shape[1]
    src = jnp.concatenate([e[0], jnp.zeros((ep - E,), jnp.int32)])
    dst = jnp.concatenate([e[1], jnp.full((ep - E,), n, jnp.int32)])
    return src.reshape(-1, _LANES), dst.reshape(-1, _LANES)


def kernel(x0, x1, x2, edge_index0, edge_index1, edge_index2,
           W0_0, W0_1, W0_2, W0_3, W0_4, W1_0, W1_1, W1_2, W1_3,
           W2_0, W2_1, W2_2, W2_3):
    N = x0.shape[0]
    E = edge_index0.shape[1]
    rw = 8 * _ceil(E, _NW * _LANES * 8)
    ep = rw * _NW * _LANES

    e0, e1, e2 = [_pad_edges(e, ep, N)
                  for e in (edge_index0, edge_index1, edge_index2)]
    z16 = jnp.zeros((_ZCH, 16), _F32)
    z8 = jnp.zeros((_ZCH, 8), _F32)
    eye8 = jnp.eye(8, dtype=_F32)

    agg16 = _make_sc_agg(N, rw, 16)
    agg8 = _make_sc_agg(N, rw, 8)
    deg = _make_sc_deg(N, rw)

    ones_t = jnp.ones((_LANES, 8), _F32)
    degs = [deg(dm, ones_t, z8) for _, dm in (e0, e1, e2)]

    p8 = [agg16(lax.slice_in_dim(x0, 16 * k, 16 * (k + 1), axis=1),
                e0[0], e0[1], z16) for k in range(8)]

    ts = _tc1(p8, degs, x1, x2, W0_0, W0_1, W1_0, W2_0, N)

    ev = (e0, e0, e1, e1, e2, e2)
    q = [agg16(ts[i], ev[i][0], ev[i][1], z16) for i in range(6)]
    u = _tc_fuse32(q, degs, (W0_2, W1_1, W2_1), N)

    r = [agg16(u[i], (e0, e1, e2)[i][0], (e0, e1, e2)[i][1], z16)
         for i in range(3)]
    v = _tc_fuse(r, degs, (W0_3, W1_2, W2_2), N, fin=16)

    sarr = [agg8(v[i], (e0, e1, e2)[i][0], (e0, e1, e2)[i][1], z8)
            for i in range(3)]
    z = _tc_fuse(sarr, degs, (eye8, eye8, eye8), N, fin=8)

    t = [agg8(z[i], (e0, e1, e2)[i][0], (e0, e1, e2)[i][1], z8)
         for i in range(3)]
    return _tc_final(t, degs, (W0_4, W1_3, W2_3), N)
